# baseline (speedup 1.0000x reference)
import functools

import jax
import jax.numpy as jnp
from jax import lax
from jax.experimental import pallas as pl
from jax.experimental.pallas import tpu as pltpu
from jax.experimental.pallas import tpu_sc as plsc

DIM = 1024
HIDDEN = 2048
E = 8
TOPK = 2
N = 2048
S = N * TOPK
BM = 256
NT = (S + E * BM) // BM
S_MAX = NT * BM

NC = 2
NS = 16
NW = NC * NS


@functools.cache
def _sc_mesh():
    return plsc.VectorSubcoreMesh(core_axis_name="c", subcore_axis_name="s",
                                  num_cores=NC, num_subcores=NS)


def _gate_body(x_ref, gw_ref, d_ref, e_ref, w1_ref, w2_ref):
    x = x_ref[...]
    gw = gw_ref[...]
    logits = lax.dot_general(x, gw, (((1,), (1,)), ((), ())),
                             preferred_element_type=jnp.float32)
    iota = lax.broadcasted_iota(jnp.int32, logits.shape, 1)
    m1 = jnp.max(logits, axis=1, keepdims=True)
    i1 = jnp.min(jnp.where(logits == m1, iota, E), axis=1, keepdims=True)
    masked = jnp.where(iota == i1, -jnp.inf, logits)
    m2 = jnp.max(masked, axis=1, keepdims=True)
    i2 = jnp.min(jnp.where(masked == m2, iota, E), axis=1, keepdims=True)
    e2 = jnp.exp(m2 - m1)
    denom = 1.0 + e2
    w2 = e2 / denom
    w2 = jnp.where(d_ref[0] != 0, 0.0, w2)
    w1 = 1.0 - w2
    e_ref[...] = jnp.concatenate([i1, i2], axis=1)
    w1_ref[...] = jnp.broadcast_to(w1, (N, 16))
    w2_ref[...] = jnp.broadcast_to(w2, (N, 16))


def _gate(x, gate_w, dflag):
    return pl.pallas_call(
        _gate_body,
        in_specs=[
            pl.BlockSpec((N, DIM), lambda: (0, 0)),
            pl.BlockSpec((E, DIM), lambda: (0, 0)),
            pl.BlockSpec(memory_space=pltpu.SMEM),
        ],
        out_specs=[
            pl.BlockSpec((N, TOPK), lambda: (0, 0)),
            pl.BlockSpec((N, 16), lambda: (0, 0)),
            pl.BlockSpec((N, 16), lambda: (0, 0)),
        ],
        out_shape=[
            jax.ShapeDtypeStruct((N, TOPK), jnp.int32),
            jax.ShapeDtypeStruct((N, 16), jnp.float32),
            jax.ShapeDtypeStruct((N, 16), jnp.float32),
        ],
    )(x, gate_w, dflag)


def _route_body(e_ref, dest_ref, te_ref):
    ev = e_ref[...]
    r128 = lax.broadcasted_iota(jnp.int32, (128, 128), 0)
    c128 = lax.broadcasted_iota(jnp.int32, (128, 128), 1)
    m_lane = (r128 < c128).astype(jnp.float32)
    r32 = lax.broadcasted_iota(jnp.int32, (32, 32), 0)
    c32 = lax.broadcasted_iota(jnp.int32, (32, 32), 1)
    m_row = (c32 < r32).astype(jnp.float32)

    within = jnp.zeros((32, 128), jnp.float32)
    masks = []
    cnts = []
    for e in range(E):
        m = (ev == e).astype(jnp.float32)
        rowpart = lax.dot_general(m, m_lane, (((1,), (0,)), ((), ())),
                                  preferred_element_type=jnp.float32)
        rowsum = jnp.sum(m, axis=1, keepdims=True)
        rowcarry = lax.dot_general(m_row, rowsum, (((1,), (0,)), ((), ())),
                                   preferred_element_type=jnp.float32)
        within = within + m * (rowpart + rowcarry)
        masks.append(m)
        cnts.append(jnp.sum(rowsum).astype(jnp.int32))

    pad_off = []
    acc = jnp.int32(0)
    for e in range(E):
        pad_off.append(acc)
        acc = acc + ((cnts[e] + BM - 1) // BM) * BM
    pad_end = pad_off[1:] + [acc]

    sel = jnp.zeros((32, 128), jnp.float32)
    for e in range(E):
        sel = sel + masks[e] * pad_off[e].astype(jnp.float32)
    dest_ref[...] = (sel + within).astype(jnp.int32)

    t_iota = lax.broadcasted_iota(jnp.int32, (1, 128), 1)
    te = jnp.zeros((1, 128), jnp.int32)
    for e in range(E):
        te = te + (pad_end[e] <= t_iota * BM).astype(jnp.int32)
    last_e = jnp.int32(0)
    for e in range(E):
        last_e = jnp.where(cnts[e] > 0, jnp.int32(e), last_e)
    te = jnp.minimum(te, last_e)
    valid = (t_iota * BM < acc).astype(jnp.int32)
    r8 = lax.broadcasted_iota(jnp.int32, (8, 128), 0)
    te_ref[...] = jnp.where(r8 == 1, jnp.broadcast_to(valid, (8, 128)),
                            jnp.broadcast_to(te, (8, 128)))


def _route(e2d):
    return pl.pallas_call(
        _route_body,
        in_specs=[pl.BlockSpec((32, 128), lambda: (0, 0))],
        out_specs=[
            pl.BlockSpec((32, 128), lambda: (0, 0)),
            pl.BlockSpec((8, 128), lambda: (0, 0)),
        ],
        out_shape=[
            jax.ShapeDtypeStruct((32, 128), jnp.int32),
            jax.ShapeDtypeStruct((8, 128), jnp.int32),
        ],
    )(e2d)


_DCH = 32


@functools.cache
def _build_dispatch():
    nch = S // (NW * _DCH)

    @functools.partial(
        pl.kernel,
        mesh=_sc_mesh(),
        out_type=jax.ShapeDtypeStruct((S_MAX, DIM), jnp.float32),
        scratch_types=[
            pltpu.VMEM((nch, _DCH), jnp.int32),
            pltpu.VMEM((nch, _DCH), jnp.int32),
            pltpu.VMEM((_DCH, DIM), jnp.float32),
            pltpu.VMEM((_DCH, DIM), jnp.float32),
            pltpu.SemaphoreType.DMA,
            pltpu.SemaphoreType.DMA,
        ],
    )
    def _dispatch(x_hbm, tok_hbm, dst_hbm, xs_hbm,
                  tok_v, dst_v, rows_a, rows_b, semg, sems):
        wid = lax.axis_index("s") * NC + lax.axis_index("c")
        pltpu.sync_copy(tok_hbm.at[wid], tok_v)
        pltpu.sync_copy(dst_hbm.at[wid], dst_v)
        bufs = (rows_a, rows_b)
        gq, sq = {}, {}
        for c in range(min(2, nch)):
            gq[c] = pltpu.async_copy(x_hbm.at[tok_v.at[c]], bufs[c % 2], semg)
        for c in range(nch):
            gq[c].wait()
            sq[c] = pltpu.async_copy(bufs[c % 2], xs_hbm.at[dst_v.at[c]], sems)
            if c + 2 < nch:
                sq[c].wait()
                gq[c + 2] = pltpu.async_copy(
                    x_hbm.at[tok_v.at[c + 2]], bufs[c % 2], semg)
        for c in range(max(0, nch - 2), nch):
            sq[c].wait()

    return _dispatch


def _ffn_body(eid_ref, val_ref, x_ref, w0_ref, w2_ref, w1_ref, y_ref):
    t = pl.program_id(0)

    @pl.when(val_ref[t] != 0)
    def _():
        x = x_ref[...].astype(jnp.bfloat16)
        a = lax.dot_general(x, w0_ref[0, 0], (((1,), (1,)), ((), ())),
                            preferred_element_type=jnp.float32)
        g = lax.dot_general(x, w2_ref[0, 0], (((1,), (1,)), ((), ())),
                            preferred_element_type=jnp.float32)
        h = (a / (1.0 + jnp.exp(-a))) * g
        y = lax.dot_general(h.astype(jnp.bfloat16), w1_ref[0, 0],
                            (((1,), (0,)), ((), ())),
                            preferred_element_type=jnp.float32)
        y_ref[...] = y


def _ffn(tile_eid, tile_valid, xs, expert_w):
    grid_spec = pltpu.PrefetchScalarGridSpec(
        num_scalar_prefetch=2,
        grid=(NT,),
        in_specs=[
            pl.BlockSpec((BM, DIM), lambda t, eid, val: (t, 0)),
            pl.BlockSpec((1, 1, HIDDEN, DIM),
                         lambda t, eid, val: (eid[t], 0, 0, 0)),
            pl.BlockSpec((1, 1, HIDDEN, DIM),
                         lambda t, eid, val: (eid[t], 2, 0, 0)),
            pl.BlockSpec((1, 1, HIDDEN, DIM),
                         lambda t, eid, val: (eid[t], 1, 0, 0)),
        ],
        out_specs=pl.BlockSpec((BM, DIM), lambda t, eid, val: (t, 0)),
    )
    return pl.pallas_call(
        _ffn_body,
        grid_spec=grid_spec,
        out_shape=jax.ShapeDtypeStruct((S_MAX, DIM), jnp.float32),
    )(tile_eid, tile_valid, xs, expert_w, expert_w, expert_w)


_CCH = 16


@functools.cache
def _build_combine():
    nch = N // (NW * _CCH)
    tpw = N // NW

    @functools.partial(
        pl.kernel,
        mesh=_sc_mesh(),
        out_type=jax.ShapeDtypeStruct((N, DIM), jnp.float32),
        scratch_types=[
            pltpu.VMEM((nch, _CCH), jnp.int32),
            pltpu.VMEM((nch, _CCH), jnp.int32),
            pltpu.VMEM((tpw, 16), jnp.float32),
            pltpu.VMEM((tpw, 16), jnp.float32),
            pltpu.VMEM((_CCH, DIM), jnp.float32),
            pltpu.VMEM((_CCH, DIM), jnp.float32),
            pltpu.VMEM((_CCH, DIM), jnp.float32),
            pltpu.VMEM((_CCH, DIM), jnp.float32),
            pltpu.SemaphoreType.DMA,
            pltpu.SemaphoreType.DMA,
        ],
    )
    def _combine(y_hbm, d1_hbm, d2_hbm, w1_hbm, w2_hbm, out_hbm,
                 i1_v, i2_v, w1_v, w2_v, r1a, r2a, r1b, r2b, semg, semw):
        wid = lax.axis_index("s") * NC + lax.axis_index("c")
        base = wid * tpw
        pltpu.sync_copy(d1_hbm.at[wid], i1_v)
        pltpu.sync_copy(d2_hbm.at[wid], i2_v)
        pltpu.sync_copy(w1_hbm.at[pl.ds(base, tpw)], w1_v)
        pltpu.sync_copy(w2_hbm.at[pl.ds(base, tpw)], w2_v)
        bufs = ((r1a, r2a), (r1b, r2b))

        def gather(c):
            b1, b2 = bufs[c % 2]
            g1 = pltpu.async_copy(y_hbm.at[i1_v.at[c]], b1, semg)
            g2 = pltpu.async_copy(y_hbm.at[i2_v.at[c]], b2, semg)
            return g1, g2

        gq, wq = {}, {}
        for c in range(min(2, nch)):
            gq[c] = gather(c)
        for c in range(nch):
            g1, g2 = gq[c]
            g1.wait()
            g2.wait()
            b1, b2 = bufs[c % 2]

            def row_comb(i, carry):
                wb1 = w1_v[c * _CCH + i, :]
                wb2 = w2_v[c * _CCH + i, :]
                for cc in range(DIM // 16):
                    sl = pl.ds(cc * 16, 16)
                    b1[i, sl] = b1[i, sl] * wb1 + b2[i, sl] * wb2
                return carry

            lax.fori_loop(0, _CCH, row_comb, 0)
            wq[c] = pltpu.async_copy(
                b1, out_hbm.at[pl.ds(base + c * _CCH, _CCH)], semw)
            if c + 2 < nch:
                wq[c].wait()
                gq[c + 2] = gather(c + 2)
        for c in range(max(0, nch - 2), nch):
            wq[c].wait()

    return _combine


def kernel(inputs, gate_w, expert_w, drafting=False):
    x = inputs
    dflag = jnp.asarray(drafting, jnp.int32).reshape((1,))
    e2, w1x, w2x = _gate(x, gate_w, dflag)

    e_flat = e2.reshape(S)
    dest2d, te = _route(e_flat.reshape(32, 128))
    dest = dest2d.reshape(S)
    tile_eid = te[0, :NT]
    tile_valid = te[1, :NT]

    tok_flat = (jnp.arange(S, dtype=jnp.int32) // TOPK)
    xs = _build_dispatch()(
        x,
        tok_flat.reshape(NW, -1, _DCH),
        dest.reshape(NW, -1, _DCH),
    )

    y_s = _ffn(tile_eid, tile_valid, xs, expert_w.astype(jnp.bfloat16))

    d_pair = dest.reshape(N, TOPK)
    return _build_combine()(
        y_s,
        d_pair[:, 0].reshape(NW, -1, _CCH),
        d_pair[:, 1].reshape(NW, -1, _CCH),
        w1x, w2x)


# speedup vs baseline: 1.0003x; 1.0003x over previous
import functools

import jax
import jax.numpy as jnp
from jax import lax
from jax.experimental import pallas as pl
from jax.experimental.pallas import tpu as pltpu
from jax.experimental.pallas import tpu_sc as plsc

DIM = 1024
HIDDEN = 2048
E = 8
TOPK = 2
N = 2048
S = N * TOPK
BM = 256
NT = (S + E * BM) // BM
S_MAX = NT * BM

NC = 2
NS = 16
NW = NC * NS


@functools.cache
def _sc_mesh():
    return plsc.VectorSubcoreMesh(core_axis_name="c", subcore_axis_name="s",
                                  num_cores=NC, num_subcores=NS)


def _gate_body(x_ref, gw_ref, d_ref, e_ref, w1_ref, w2_ref):
    x = x_ref[...]
    gw = gw_ref[...]
    logits = lax.dot_general(x, gw, (((1,), (1,)), ((), ())),
                             preferred_element_type=jnp.float32)
    iota = lax.broadcasted_iota(jnp.int32, logits.shape, 1)
    m1 = jnp.max(logits, axis=1, keepdims=True)
    i1 = jnp.min(jnp.where(logits == m1, iota, E), axis=1, keepdims=True)
    masked = jnp.where(iota == i1, -jnp.inf, logits)
    m2 = jnp.max(masked, axis=1, keepdims=True)
    i2 = jnp.min(jnp.where(masked == m2, iota, E), axis=1, keepdims=True)
    e2 = jnp.exp(m2 - m1)
    denom = 1.0 + e2
    w2 = e2 / denom
    w2 = jnp.where(d_ref[0] != 0, 0.0, w2)
    w1 = 1.0 - w2
    e_ref[...] = jnp.concatenate([i1, i2], axis=1)
    w1_ref[...] = jnp.broadcast_to(w1, (N, 16))
    w2_ref[...] = jnp.broadcast_to(w2, (N, 16))


def _gate(x, gate_w, dflag):
    return pl.pallas_call(
        _gate_body,
        in_specs=[
            pl.BlockSpec((N, DIM), lambda: (0, 0)),
            pl.BlockSpec((E, DIM), lambda: (0, 0)),
            pl.BlockSpec(memory_space=pltpu.SMEM),
        ],
        out_specs=[
            pl.BlockSpec((N, TOPK), lambda: (0, 0)),
            pl.BlockSpec((N, 16), lambda: (0, 0)),
            pl.BlockSpec((N, 16), lambda: (0, 0)),
        ],
        out_shape=[
            jax.ShapeDtypeStruct((N, TOPK), jnp.int32),
            jax.ShapeDtypeStruct((N, 16), jnp.float32),
            jax.ShapeDtypeStruct((N, 16), jnp.float32),
        ],
    )(x, gate_w, dflag)


def _route_body(e_ref, dest_ref, te_ref):
    ev = e_ref[...]
    r128 = lax.broadcasted_iota(jnp.int32, (128, 128), 0)
    c128 = lax.broadcasted_iota(jnp.int32, (128, 128), 1)
    m_lane = (r128 < c128).astype(jnp.float32)
    r32 = lax.broadcasted_iota(jnp.int32, (32, 32), 0)
    c32 = lax.broadcasted_iota(jnp.int32, (32, 32), 1)
    m_row = (c32 < r32).astype(jnp.float32)

    within = jnp.zeros((32, 128), jnp.float32)
    masks = []
    cnts = []
    for e in range(E):
        m = (ev == e).astype(jnp.float32)
        rowpart = lax.dot_general(m, m_lane, (((1,), (0,)), ((), ())),
                                  preferred_element_type=jnp.float32)
        rowsum = jnp.sum(m, axis=1, keepdims=True)
        rowcarry = lax.dot_general(m_row, rowsum, (((1,), (0,)), ((), ())),
                                   preferred_element_type=jnp.float32)
        within = within + m * (rowpart + rowcarry)
        masks.append(m)
        cnts.append(jnp.sum(rowsum).astype(jnp.int32))

    pad_off = []
    acc = jnp.int32(0)
    for e in range(E):
        pad_off.append(acc)
        acc = acc + ((cnts[e] + BM - 1) // BM) * BM
    pad_end = pad_off[1:] + [acc]

    sel = jnp.zeros((32, 128), jnp.float32)
    for e in range(E):
        sel = sel + masks[e] * pad_off[e].astype(jnp.float32)
    dest_ref[...] = (sel + within).astype(jnp.int32)

    t_iota = lax.broadcasted_iota(jnp.int32, (1, 128), 1)
    te = jnp.zeros((1, 128), jnp.int32)
    for e in range(E):
        te = te + (pad_end[e] <= t_iota * BM).astype(jnp.int32)
    last_e = jnp.int32(0)
    for e in range(E):
        last_e = jnp.where(cnts[e] > 0, jnp.int32(e), last_e)
    te = jnp.minimum(te, last_e)
    valid = (t_iota * BM < acc).astype(jnp.int32)
    r8 = lax.broadcasted_iota(jnp.int32, (8, 128), 0)
    te_ref[...] = jnp.where(r8 == 1, jnp.broadcast_to(valid, (8, 128)),
                            jnp.broadcast_to(te, (8, 128)))


def _route(e2d):
    return pl.pallas_call(
        _route_body,
        in_specs=[pl.BlockSpec((32, 128), lambda: (0, 0))],
        out_specs=[
            pl.BlockSpec((32, 128), lambda: (0, 0)),
            pl.BlockSpec((8, 128), lambda: (0, 0)),
        ],
        out_shape=[
            jax.ShapeDtypeStruct((32, 128), jnp.int32),
            jax.ShapeDtypeStruct((8, 128), jnp.int32),
        ],
    )(e2d)


_DCH = 32


@functools.cache
def _build_dispatch():
    nch = S // (NW * _DCH)

    @functools.partial(
        pl.kernel,
        mesh=_sc_mesh(),
        out_type=jax.ShapeDtypeStruct((S_MAX, DIM), jnp.float32),
        scratch_types=[
            pltpu.VMEM((nch, _DCH), jnp.int32),
            pltpu.VMEM((nch, _DCH), jnp.int32),
            pltpu.VMEM((_DCH, DIM), jnp.float32),
            pltpu.VMEM((_DCH, DIM), jnp.float32),
            pltpu.SemaphoreType.DMA,
            pltpu.SemaphoreType.DMA,
        ],
    )
    def _dispatch(x_hbm, tok_hbm, dst_hbm, xs_hbm,
                  tok_v, dst_v, rows_a, rows_b, semg, sems):
        wid = lax.axis_index("s") * NC + lax.axis_index("c")
        pltpu.sync_copy(tok_hbm.at[wid], tok_v)
        pltpu.sync_copy(dst_hbm.at[wid], dst_v)
        bufs = (rows_a, rows_b)
        gq, sq = {}, {}
        for c in range(min(2, nch)):
            gq[c] = pltpu.async_copy(x_hbm.at[tok_v.at[c]], bufs[c % 2], semg)
        for c in range(nch):
            gq[c].wait()
            sq[c] = pltpu.async_copy(bufs[c % 2], xs_hbm.at[dst_v.at[c]], sems)
            if c + 2 < nch:
                sq[c].wait()
                gq[c + 2] = pltpu.async_copy(
                    x_hbm.at[tok_v.at[c + 2]], bufs[c % 2], semg)
        for c in range(max(0, nch - 2), nch):
            sq[c].wait()

    return _dispatch


def _ffn_body(eid_ref, val_ref, x_ref, w0a_ref, w0b_ref, w2a_ref, w2b_ref,
              w1a_ref, w1b_ref, y_ref):
    t = pl.program_id(0)

    @pl.when(val_ref[t] != 0)
    def _():
        x = x_ref[...].astype(jnp.bfloat16)

        def dot_t(u, wref):
            return lax.dot_general(
                u, wref[0, 0].astype(jnp.bfloat16), (((1,), (1,)), ((), ())),
                preferred_element_type=jnp.float32)

        def dot_n(u, wref):
            return lax.dot_general(
                u.astype(jnp.bfloat16), wref[0, 0].astype(jnp.bfloat16),
                (((1,), (0,)), ((), ())), preferred_element_type=jnp.float32)

        y = jnp.zeros((BM, DIM), jnp.float32)
        for w0h, w2h, w1h in ((w0a_ref, w2a_ref, w1a_ref),
                              (w0b_ref, w2b_ref, w1b_ref)):
            a = dot_t(x, w0h)
            g = dot_t(x, w2h)
            h = (a / (1.0 + jnp.exp(-a))) * g
            y = y + dot_n(h, w1h)
        y_ref[...] = y


def _ffn(tile_eid, tile_valid, xs, expert_w):
    grid_spec = pltpu.PrefetchScalarGridSpec(
        num_scalar_prefetch=2,
        grid=(NT,),
        in_specs=[pl.BlockSpec((BM, DIM), lambda t, eid, val: (t, 0))] + [
            pl.BlockSpec(
                (1, 1, HIDDEN // 2, DIM),
                functools.partial(
                    lambda t, eid, val, _j, _h: (eid[t], _j, _h, 0),
                    _j=j, _h=hh))
            for j in (0, 2, 1) for hh in (0, 1)
        ],
        out_specs=pl.BlockSpec((BM, DIM), lambda t, eid, val: (t, 0)),
    )
    return pl.pallas_call(
        _ffn_body,
        grid_spec=grid_spec,
        out_shape=jax.ShapeDtypeStruct((S_MAX, DIM), jnp.float32),
    )(tile_eid, tile_valid, xs, *([expert_w] * 6))


_CCH = 16


@functools.cache
def _build_combine():
    nch = N // (NW * _CCH)
    tpw = N // NW

    @functools.partial(
        pl.kernel,
        mesh=_sc_mesh(),
        out_type=jax.ShapeDtypeStruct((N, DIM), jnp.float32),
        scratch_types=[
            pltpu.VMEM((nch, _CCH), jnp.int32),
            pltpu.VMEM((nch, _CCH), jnp.int32),
            pltpu.VMEM((tpw, 16), jnp.float32),
            pltpu.VMEM((tpw, 16), jnp.float32),
            pltpu.VMEM((_CCH, DIM), jnp.float32),
            pltpu.VMEM((_CCH, DIM), jnp.float32),
            pltpu.VMEM((_CCH, DIM), jnp.float32),
            pltpu.VMEM((_CCH, DIM), jnp.float32),
            pltpu.SemaphoreType.DMA,
            pltpu.SemaphoreType.DMA,
        ],
    )
    def _combine(y_hbm, d1_hbm, d2_hbm, w1_hbm, w2_hbm, out_hbm,
                 i1_v, i2_v, w1_v, w2_v, r1a, r2a, r1b, r2b, semg, semw):
        wid = lax.axis_index("s") * NC + lax.axis_index("c")
        base = wid * tpw
        pltpu.sync_copy(d1_hbm.at[wid], i1_v)
        pltpu.sync_copy(d2_hbm.at[wid], i2_v)
        pltpu.sync_copy(w1_hbm.at[pl.ds(base, tpw)], w1_v)
        pltpu.sync_copy(w2_hbm.at[pl.ds(base, tpw)], w2_v)
        bufs = ((r1a, r2a), (r1b, r2b))

        def gather(c):
            b1, b2 = bufs[c % 2]
            g1 = pltpu.async_copy(y_hbm.at[i1_v.at[c]], b1, semg)
            g2 = pltpu.async_copy(y_hbm.at[i2_v.at[c]], b2, semg)
            return g1, g2

        gq, wq = {}, {}
        for c in range(min(2, nch)):
            gq[c] = gather(c)
        for c in range(nch):
            g1, g2 = gq[c]
            g1.wait()
            g2.wait()
            b1, b2 = bufs[c % 2]

            def row_comb(i, carry):
                wb1 = w1_v[c * _CCH + i, :]
                wb2 = w2_v[c * _CCH + i, :]
                for cc in range(DIM // 16):
                    sl = pl.ds(cc * 16, 16)
                    b1[i, sl] = b1[i, sl] * wb1 + b2[i, sl] * wb2
                return carry

            lax.fori_loop(0, _CCH, row_comb, 0)
            wq[c] = pltpu.async_copy(
                b1, out_hbm.at[pl.ds(base + c * _CCH, _CCH)], semw)
            if c + 2 < nch:
                wq[c].wait()
                gq[c + 2] = gather(c + 2)
        for c in range(max(0, nch - 2), nch):
            wq[c].wait()

    return _combine


def kernel(inputs, gate_w, expert_w, drafting=False):
    x = inputs
    dflag = jnp.asarray(drafting, jnp.int32).reshape((1,))
    e2, w1x, w2x = _gate(x, gate_w, dflag)

    e_flat = e2.reshape(S)
    dest2d, te = _route(e_flat.reshape(32, 128))
    dest = dest2d.reshape(S)
    tile_eid = te[0, :NT]
    tile_valid = te[1, :NT]

    tok_flat = (jnp.arange(S, dtype=jnp.int32) // TOPK)
    xs = _build_dispatch()(
        x,
        tok_flat.reshape(NW, -1, _DCH),
        dest.reshape(NW, -1, _DCH),
    )

    y_s = _ffn(tile_eid, tile_valid, xs, expert_w.astype(jnp.bfloat16))

    d_pair = dest.reshape(N, TOPK)
    return _build_combine()(
        y_s,
        d_pair[:, 0].reshape(NW, -1, _CCH),
        d_pair[:, 1].reshape(NW, -1, _CCH),
        w1x, w2x)


# speedup vs baseline: 1.3261x; 1.3257x over previous
import functools

import jax
import jax.numpy as jnp
from jax import lax
from jax.experimental import pallas as pl
from jax.experimental.pallas import tpu as pltpu
from jax.experimental.pallas import tpu_sc as plsc

DIM = 1024
HIDDEN = 2048
E = 8
TOPK = 2
N = 2048
S = N * TOPK
BM = 256
NT = (S + E * BM) // BM
S_MAX = NT * BM

NC = 2
NS = 16
NW = NC * NS


@functools.cache
def _sc_mesh():
    return plsc.VectorSubcoreMesh(core_axis_name="c", subcore_axis_name="s",
                                  num_cores=NC, num_subcores=NS)


def _gate_body(x_ref, gw_ref, d_ref, e_ref, w1_ref, w2_ref):
    x = x_ref[...]
    gw = gw_ref[...]
    logits = lax.dot_general(x, gw, (((1,), (1,)), ((), ())),
                             preferred_element_type=jnp.float32)
    iota = lax.broadcasted_iota(jnp.int32, logits.shape, 1)
    m1 = jnp.max(logits, axis=1, keepdims=True)
    i1 = jnp.min(jnp.where(logits == m1, iota, E), axis=1, keepdims=True)
    masked = jnp.where(iota == i1, -jnp.inf, logits)
    m2 = jnp.max(masked, axis=1, keepdims=True)
    i2 = jnp.min(jnp.where(masked == m2, iota, E), axis=1, keepdims=True)
    e2 = jnp.exp(m2 - m1)
    denom = 1.0 + e2
    w2 = e2 / denom
    w2 = jnp.where(d_ref[0] != 0, 0.0, w2)
    w1 = 1.0 - w2
    e_ref[...] = jnp.concatenate([i1, i2], axis=1)
    w1_ref[...] = jnp.broadcast_to(w1, (N, 16))
    w2_ref[...] = jnp.broadcast_to(w2, (N, 16))


def _gate(x, gate_w, dflag):
    return pl.pallas_call(
        _gate_body,
        in_specs=[
            pl.BlockSpec((N, DIM), lambda: (0, 0)),
            pl.BlockSpec((E, DIM), lambda: (0, 0)),
            pl.BlockSpec(memory_space=pltpu.SMEM),
        ],
        out_specs=[
            pl.BlockSpec((N, TOPK), lambda: (0, 0)),
            pl.BlockSpec((N, 16), lambda: (0, 0)),
            pl.BlockSpec((N, 16), lambda: (0, 0)),
        ],
        out_shape=[
            jax.ShapeDtypeStruct((N, TOPK), jnp.int32),
            jax.ShapeDtypeStruct((N, 16), jnp.float32),
            jax.ShapeDtypeStruct((N, 16), jnp.float32),
        ],
    )(x, gate_w, dflag)


def _route_body(e_ref, dest_ref, te_ref):
    ev = e_ref[...]
    r128 = lax.broadcasted_iota(jnp.int32, (128, 128), 0)
    c128 = lax.broadcasted_iota(jnp.int32, (128, 128), 1)
    m_lane = (r128 < c128).astype(jnp.float32)
    r32 = lax.broadcasted_iota(jnp.int32, (32, 32), 0)
    c32 = lax.broadcasted_iota(jnp.int32, (32, 32), 1)
    m_row = (c32 < r32).astype(jnp.float32)

    within = jnp.zeros((32, 128), jnp.float32)
    masks = []
    cnts = []
    for e in range(E):
        m = (ev == e).astype(jnp.float32)
        rowpart = lax.dot_general(m, m_lane, (((1,), (0,)), ((), ())),
                                  preferred_element_type=jnp.float32)
        rowsum = jnp.sum(m, axis=1, keepdims=True)
        rowcarry = lax.dot_general(m_row, rowsum, (((1,), (0,)), ((), ())),
                                   preferred_element_type=jnp.float32)
        within = within + m * (rowpart + rowcarry)
        masks.append(m)
        cnts.append(jnp.sum(rowsum).astype(jnp.int32))

    pad_off = []
    acc = jnp.int32(0)
    for e in range(E):
        pad_off.append(acc)
        acc = acc + ((cnts[e] + BM - 1) // BM) * BM
    pad_end = pad_off[1:] + [acc]

    sel = jnp.zeros((32, 128), jnp.float32)
    for e in range(E):
        sel = sel + masks[e] * pad_off[e].astype(jnp.float32)
    dest_ref[...] = (sel + within).astype(jnp.int32)

    t_iota = lax.broadcasted_iota(jnp.int32, (1, 128), 1)
    te = jnp.zeros((1, 128), jnp.int32)
    for e in range(E):
        te = te + (pad_end[e] <= t_iota * BM).astype(jnp.int32)
    last_e = jnp.int32(0)
    for e in range(E):
        last_e = jnp.where(cnts[e] > 0, jnp.int32(e), last_e)
    te = jnp.minimum(te, last_e)
    valid = (t_iota * BM < acc).astype(jnp.int32)
    r8 = lax.broadcasted_iota(jnp.int32, (8, 128), 0)
    te_ref[...] = jnp.where(r8 == 1, jnp.broadcast_to(valid, (8, 128)),
                            jnp.broadcast_to(te, (8, 128)))


def _route(e2d):
    return pl.pallas_call(
        _route_body,
        in_specs=[pl.BlockSpec((32, 128), lambda: (0, 0))],
        out_specs=[
            pl.BlockSpec((32, 128), lambda: (0, 0)),
            pl.BlockSpec((8, 128), lambda: (0, 0)),
        ],
        out_shape=[
            jax.ShapeDtypeStruct((32, 128), jnp.int32),
            jax.ShapeDtypeStruct((8, 128), jnp.int32),
        ],
    )(e2d)


_DCH = 32


@functools.cache
def _build_dispatch():
    nch = S // (NW * _DCH)

    @functools.partial(
        pl.kernel,
        mesh=_sc_mesh(),
        out_type=jax.ShapeDtypeStruct((S_MAX, DIM), jnp.float32),
        scratch_types=[
            pltpu.VMEM((nch, _DCH), jnp.int32),
            pltpu.VMEM((nch, _DCH), jnp.int32),
            pltpu.VMEM((_DCH, DIM), jnp.float32),
            pltpu.VMEM((_DCH, DIM), jnp.float32),
            pltpu.SemaphoreType.DMA,
            pltpu.SemaphoreType.DMA,
        ],
    )
    def _dispatch(x_hbm, tok_hbm, dst_hbm, xs_hbm,
                  tok_v, dst_v, rows_a, rows_b, semg, sems):
        wid = lax.axis_index("s") * NC + lax.axis_index("c")
        pltpu.sync_copy(tok_hbm.at[wid], tok_v)
        pltpu.sync_copy(dst_hbm.at[wid], dst_v)
        bufs = (rows_a, rows_b)
        gq, sq = {}, {}
        for c in range(min(2, nch)):
            gq[c] = pltpu.async_copy(x_hbm.at[tok_v.at[c]], bufs[c % 2], semg)
        for c in range(nch):
            gq[c].wait()
            sq[c] = pltpu.async_copy(bufs[c % 2], xs_hbm.at[dst_v.at[c]], sems)
            if c + 2 < nch:
                sq[c].wait()
                gq[c + 2] = pltpu.async_copy(
                    x_hbm.at[tok_v.at[c + 2]], bufs[c % 2], semg)
        for c in range(max(0, nch - 2), nch):
            sq[c].wait()

    return _dispatch


def _ffn_body(eid_ref, val_ref, x_ref, w0_ref, w2_ref, w1_ref, y_ref):
    t = pl.program_id(0)

    @pl.when(val_ref[t] != 0)
    def _():
        x = x_ref[...].astype(jnp.bfloat16)
        a = lax.dot_general(x, w0_ref[0, 0].astype(jnp.bfloat16),
                            (((1,), (1,)), ((), ())),
                            preferred_element_type=jnp.float32)
        g = lax.dot_general(x, w2_ref[0, 0].astype(jnp.bfloat16),
                            (((1,), (1,)), ((), ())),
                            preferred_element_type=jnp.float32)
        h = (a / (1.0 + jnp.exp(-a))) * g
        y = lax.dot_general(h.astype(jnp.bfloat16),
                            w1_ref[0, 0].astype(jnp.bfloat16),
                            (((1,), (0,)), ((), ())),
                            preferred_element_type=jnp.float32)
        y_ref[...] = y


def _ffn(tile_eid, tile_valid, xs, expert_w):
    grid_spec = pltpu.PrefetchScalarGridSpec(
        num_scalar_prefetch=2,
        grid=(NT,),
        in_specs=[
            pl.BlockSpec((BM, DIM), lambda t, eid, val: (t, 0)),
            pl.BlockSpec((1, 1, HIDDEN, DIM),
                         lambda t, eid, val: (eid[t], 0, 0, 0)),
            pl.BlockSpec((1, 1, HIDDEN, DIM),
                         lambda t, eid, val: (eid[t], 2, 0, 0)),
            pl.BlockSpec((1, 1, HIDDEN, DIM),
                         lambda t, eid, val: (eid[t], 1, 0, 0)),
        ],
        out_specs=pl.BlockSpec((BM, DIM), lambda t, eid, val: (t, 0)),
    )
    return pl.pallas_call(
        _ffn_body,
        grid_spec=grid_spec,
        out_shape=jax.ShapeDtypeStruct((S_MAX, DIM), jnp.float32),
    )(tile_eid, tile_valid, xs, expert_w, expert_w, expert_w)


_CCH = 16


@functools.cache
def _build_combine():
    nch = N // (NW * _CCH)
    tpw = N // NW

    @functools.partial(
        pl.kernel,
        mesh=_sc_mesh(),
        out_type=jax.ShapeDtypeStruct((N, DIM), jnp.float32),
        scratch_types=[
            pltpu.VMEM((nch, _CCH), jnp.int32),
            pltpu.VMEM((nch, _CCH), jnp.int32),
            pltpu.VMEM((tpw, 16), jnp.float32),
            pltpu.VMEM((tpw, 16), jnp.float32),
            pltpu.VMEM((_CCH, DIM), jnp.float32),
            pltpu.VMEM((_CCH, DIM), jnp.float32),
            pltpu.VMEM((_CCH, DIM), jnp.float32),
            pltpu.VMEM((_CCH, DIM), jnp.float32),
            pltpu.SemaphoreType.DMA,
            pltpu.SemaphoreType.DMA,
        ],
    )
    def _combine(y_hbm, d1_hbm, d2_hbm, w1_hbm, w2_hbm, out_hbm,
                 i1_v, i2_v, w1_v, w2_v, r1a, r2a, r1b, r2b, semg, semw):
        wid = lax.axis_index("s") * NC + lax.axis_index("c")
        base = wid * tpw
        pltpu.sync_copy(d1_hbm.at[wid], i1_v)
        pltpu.sync_copy(d2_hbm.at[wid], i2_v)
        pltpu.sync_copy(w1_hbm.at[pl.ds(base, tpw)], w1_v)
        pltpu.sync_copy(w2_hbm.at[pl.ds(base, tpw)], w2_v)
        bufs = ((r1a, r2a), (r1b, r2b))

        def gather(c):
            b1, b2 = bufs[c % 2]
            g1 = pltpu.async_copy(y_hbm.at[i1_v.at[c]], b1, semg)
            g2 = pltpu.async_copy(y_hbm.at[i2_v.at[c]], b2, semg)
            return g1, g2

        gq, wq = {}, {}
        for c in range(min(2, nch)):
            gq[c] = gather(c)
        for c in range(nch):
            g1, g2 = gq[c]
            g1.wait()
            g2.wait()
            b1, b2 = bufs[c % 2]

            def row_comb(i, carry):
                wb1 = w1_v[c * _CCH + i, :]
                wb2 = w2_v[c * _CCH + i, :]
                for cc in range(DIM // 16):
                    sl = pl.ds(cc * 16, 16)
                    b1[i, sl] = b1[i, sl] * wb1 + b2[i, sl] * wb2
                return carry

            lax.fori_loop(0, _CCH, row_comb, 0)
            wq[c] = pltpu.async_copy(
                b1, out_hbm.at[pl.ds(base + c * _CCH, _CCH)], semw)
            if c + 2 < nch:
                wq[c].wait()
                gq[c + 2] = gather(c + 2)
        for c in range(max(0, nch - 2), nch):
            wq[c].wait()

    return _combine


def kernel(inputs, gate_w, expert_w, drafting=False):
    x = inputs
    dflag = jnp.asarray(drafting, jnp.int32).reshape((1,))
    e2, w1x, w2x = _gate(x, gate_w, dflag)

    e_flat = e2.reshape(S)
    dest2d, te = _route(e_flat.reshape(32, 128))
    dest = dest2d.reshape(S)
    tile_eid = te[0, :NT]
    tile_valid = te[1, :NT]

    tok_flat = (jnp.arange(S, dtype=jnp.int32) // TOPK)
    xs = _build_dispatch()(
        x,
        tok_flat.reshape(NW, -1, _DCH),
        dest.reshape(NW, -1, _DCH),
    )

    y_s = _ffn(tile_eid, tile_valid, xs, expert_w)

    d_pair = dest.reshape(N, TOPK)
    return _build_combine()(
        y_s,
        d_pair[:, 0].reshape(NW, -1, _CCH),
        d_pair[:, 1].reshape(NW, -1, _CCH),
        w1x, w2x)


# speedup vs baseline: 1.3388x; 1.0096x over previous
import functools

import jax
import jax.numpy as jnp
from jax import lax
from jax.experimental import pallas as pl
from jax.experimental.pallas import tpu as pltpu
from jax.experimental.pallas import tpu_sc as plsc

DIM = 1024
HIDDEN = 2048
E = 8
TOPK = 2
N = 2048
S = N * TOPK
BM = 256
NT = (S + E * BM) // BM
S_MAX = NT * BM

NC = 2
NS = 16
NW = NC * NS


@functools.cache
def _sc_mesh():
    return plsc.VectorSubcoreMesh(core_axis_name="c", subcore_axis_name="s",
                                  num_cores=NC, num_subcores=NS)


def _gate_body(x_ref, gw_ref, d_ref, e_ref, w1_ref, w2_ref):
    x = x_ref[...]
    gw = gw_ref[...]
    logits = lax.dot_general(x, gw, (((1,), (1,)), ((), ())),
                             preferred_element_type=jnp.float32)
    iota = lax.broadcasted_iota(jnp.int32, logits.shape, 1)
    m1 = jnp.max(logits, axis=1, keepdims=True)
    i1 = jnp.min(jnp.where(logits == m1, iota, E), axis=1, keepdims=True)
    masked = jnp.where(iota == i1, -jnp.inf, logits)
    m2 = jnp.max(masked, axis=1, keepdims=True)
    i2 = jnp.min(jnp.where(masked == m2, iota, E), axis=1, keepdims=True)
    e2 = jnp.exp(m2 - m1)
    denom = 1.0 + e2
    w2 = e2 / denom
    w2 = jnp.where(d_ref[0] != 0, 0.0, w2)
    w1 = 1.0 - w2
    e_ref[...] = jnp.concatenate([i1, i2], axis=1)
    w1_ref[...] = jnp.broadcast_to(w1, (N, 16))
    w2_ref[...] = jnp.broadcast_to(w2, (N, 16))


def _gate(x, gate_w, dflag):
    return pl.pallas_call(
        _gate_body,
        in_specs=[
            pl.BlockSpec((N, DIM), lambda: (0, 0)),
            pl.BlockSpec((E, DIM), lambda: (0, 0)),
            pl.BlockSpec(memory_space=pltpu.SMEM),
        ],
        out_specs=[
            pl.BlockSpec((N, TOPK), lambda: (0, 0)),
            pl.BlockSpec((N, 16), lambda: (0, 0)),
            pl.BlockSpec((N, 16), lambda: (0, 0)),
        ],
        out_shape=[
            jax.ShapeDtypeStruct((N, TOPK), jnp.int32),
            jax.ShapeDtypeStruct((N, 16), jnp.float32),
            jax.ShapeDtypeStruct((N, 16), jnp.float32),
        ],
    )(x, gate_w, dflag)


def _route_body(e_ref, dest_ref, te_ref):
    ev = e_ref[...]
    r128 = lax.broadcasted_iota(jnp.int32, (128, 128), 0)
    c128 = lax.broadcasted_iota(jnp.int32, (128, 128), 1)
    m_lane = (r128 < c128).astype(jnp.float32)
    r32 = lax.broadcasted_iota(jnp.int32, (32, 32), 0)
    c32 = lax.broadcasted_iota(jnp.int32, (32, 32), 1)
    m_row = (c32 < r32).astype(jnp.float32)

    within = jnp.zeros((32, 128), jnp.float32)
    masks = []
    cnts = []
    for e in range(E):
        m = (ev == e).astype(jnp.float32)
        rowpart = lax.dot_general(m, m_lane, (((1,), (0,)), ((), ())),
                                  preferred_element_type=jnp.float32)
        rowsum = jnp.sum(m, axis=1, keepdims=True)
        rowcarry = lax.dot_general(m_row, rowsum, (((1,), (0,)), ((), ())),
                                   preferred_element_type=jnp.float32)
        within = within + m * (rowpart + rowcarry)
        masks.append(m)
        cnts.append(jnp.sum(rowsum).astype(jnp.int32))

    pad_off = []
    acc = jnp.int32(0)
    for e in range(E):
        pad_off.append(acc)
        acc = acc + ((cnts[e] + BM - 1) // BM) * BM
    pad_end = pad_off[1:] + [acc]

    sel = jnp.zeros((32, 128), jnp.float32)
    for e in range(E):
        sel = sel + masks[e] * pad_off[e].astype(jnp.float32)
    dest_ref[...] = (sel + within).astype(jnp.int32)

    t_iota = lax.broadcasted_iota(jnp.int32, (1, 128), 1)
    te = jnp.zeros((1, 128), jnp.int32)
    for e in range(E):
        te = te + (pad_end[e] <= t_iota * BM).astype(jnp.int32)
    last_e = jnp.int32(0)
    for e in range(E):
        last_e = jnp.where(cnts[e] > 0, jnp.int32(e), last_e)
    te = jnp.minimum(te, last_e)
    valid = (t_iota * BM < acc).astype(jnp.int32)
    r8 = lax.broadcasted_iota(jnp.int32, (8, 128), 0)
    te_ref[...] = jnp.where(r8 == 1, jnp.broadcast_to(valid, (8, 128)),
                            jnp.broadcast_to(te, (8, 128)))


def _route(e2d):
    return pl.pallas_call(
        _route_body,
        in_specs=[pl.BlockSpec((32, 128), lambda: (0, 0))],
        out_specs=[
            pl.BlockSpec((32, 128), lambda: (0, 0)),
            pl.BlockSpec((8, 128), lambda: (0, 0)),
        ],
        out_shape=[
            jax.ShapeDtypeStruct((32, 128), jnp.int32),
            jax.ShapeDtypeStruct((8, 128), jnp.int32),
        ],
    )(e2d)


_DCH = 32


@functools.cache
def _build_dispatch():
    nch = S // (NW * _DCH)

    @functools.partial(
        pl.kernel,
        mesh=_sc_mesh(),
        out_type=jax.ShapeDtypeStruct((S_MAX, DIM), jnp.float32),
        scratch_types=[
            pltpu.VMEM((nch, _DCH), jnp.int32),
            pltpu.VMEM((nch, _DCH), jnp.int32),
            pltpu.VMEM((_DCH, DIM), jnp.float32),
            pltpu.VMEM((_DCH, DIM), jnp.float32),
            pltpu.VMEM((_DCH, DIM), jnp.float32),
            pltpu.SemaphoreType.DMA,
            pltpu.SemaphoreType.DMA,
        ],
    )
    def _dispatch(x_hbm, tok_hbm, dst_hbm, xs_hbm,
                  tok_v, dst_v, rows_a, rows_b, rows_c, semg, sems):
        wid = lax.axis_index("s") * NC + lax.axis_index("c")
        pltpu.sync_copy(tok_hbm.at[wid], tok_v)
        pltpu.sync_copy(dst_hbm.at[wid], dst_v)
        bufs = (rows_a, rows_b, rows_c)
        nb = len(bufs)
        gq, sq = {}, {}
        for c in range(min(nb, nch)):
            gq[c] = pltpu.async_copy(x_hbm.at[tok_v.at[c]], bufs[c % nb], semg)
        for c in range(nch):
            gq[c].wait()
            sq[c] = pltpu.async_copy(bufs[c % nb], xs_hbm.at[dst_v.at[c]], sems)
            if c + nb < nch:
                sq[c].wait()
                gq[c + nb] = pltpu.async_copy(
                    x_hbm.at[tok_v.at[c + nb]], bufs[c % nb], semg)
        for c in range(max(0, nch - nb), nch):
            sq[c].wait()

    return _dispatch


def _ffn_body(eid_ref, val_ref, x_ref, w0_ref, w2_ref, w1_ref, y_ref):
    t = pl.program_id(0)

    @pl.when(val_ref[t] != 0)
    def _():
        x = x_ref[...].astype(jnp.bfloat16)
        a = lax.dot_general(x, w0_ref[0, 0].astype(jnp.bfloat16),
                            (((1,), (1,)), ((), ())),
                            preferred_element_type=jnp.float32)
        g = lax.dot_general(x, w2_ref[0, 0].astype(jnp.bfloat16),
                            (((1,), (1,)), ((), ())),
                            preferred_element_type=jnp.float32)
        h = (a / (1.0 + jnp.exp(-a))) * g
        y = lax.dot_general(h.astype(jnp.bfloat16),
                            w1_ref[0, 0].astype(jnp.bfloat16),
                            (((1,), (0,)), ((), ())),
                            preferred_element_type=jnp.float32)
        y_ref[...] = y


def _ffn(tile_eid, tile_valid, xs, expert_w):
    grid_spec = pltpu.PrefetchScalarGridSpec(
        num_scalar_prefetch=2,
        grid=(NT,),
        in_specs=[
            pl.BlockSpec((BM, DIM), lambda t, eid, val: (t, 0)),
            pl.BlockSpec((1, 1, HIDDEN, DIM),
                         lambda t, eid, val: (eid[t], 0, 0, 0)),
            pl.BlockSpec((1, 1, HIDDEN, DIM),
                         lambda t, eid, val: (eid[t], 2, 0, 0)),
            pl.BlockSpec((1, 1, HIDDEN, DIM),
                         lambda t, eid, val: (eid[t], 1, 0, 0)),
        ],
        out_specs=pl.BlockSpec((BM, DIM), lambda t, eid, val: (t, 0)),
    )
    return pl.pallas_call(
        _ffn_body,
        grid_spec=grid_spec,
        out_shape=jax.ShapeDtypeStruct((S_MAX, DIM), jnp.float32),
    )(tile_eid, tile_valid, xs, expert_w, expert_w, expert_w)


_CCH = 16


@functools.cache
def _build_combine():
    nch = N // (NW * _CCH)
    tpw = N // NW

    @functools.partial(
        pl.kernel,
        mesh=_sc_mesh(),
        out_type=jax.ShapeDtypeStruct((N, DIM), jnp.float32),
        scratch_types=[
            pltpu.VMEM((nch, _CCH), jnp.int32),
            pltpu.VMEM((nch, _CCH), jnp.int32),
            pltpu.VMEM((tpw, 16), jnp.float32),
            pltpu.VMEM((tpw, 16), jnp.float32),
            pltpu.VMEM((_CCH, DIM), jnp.float32),
            pltpu.VMEM((_CCH, DIM), jnp.float32),
            pltpu.VMEM((_CCH, DIM), jnp.float32),
            pltpu.VMEM((_CCH, DIM), jnp.float32),
            pltpu.VMEM((_CCH, DIM), jnp.float32),
            pltpu.VMEM((_CCH, DIM), jnp.float32),
            pltpu.SemaphoreType.DMA,
            pltpu.SemaphoreType.DMA,
        ],
    )
    def _combine(y_hbm, d1_hbm, d2_hbm, w1_hbm, w2_hbm, out_hbm,
                 i1_v, i2_v, w1_v, w2_v, r1a, r2a, r1b, r2b, r1c, r2c,
                 semg, semw):
        wid = lax.axis_index("s") * NC + lax.axis_index("c")
        base = wid * tpw
        pltpu.sync_copy(d1_hbm.at[wid], i1_v)
        pltpu.sync_copy(d2_hbm.at[wid], i2_v)
        pltpu.sync_copy(w1_hbm.at[pl.ds(base, tpw)], w1_v)
        pltpu.sync_copy(w2_hbm.at[pl.ds(base, tpw)], w2_v)
        bufs = ((r1a, r2a), (r1b, r2b), (r1c, r2c))
        nb = len(bufs)

        def gather(c):
            b1, b2 = bufs[c % nb]
            g1 = pltpu.async_copy(y_hbm.at[i1_v.at[c]], b1, semg)
            g2 = pltpu.async_copy(y_hbm.at[i2_v.at[c]], b2, semg)
            return g1, g2

        gq, wq = {}, {}
        for c in range(min(nb, nch)):
            gq[c] = gather(c)
        for c in range(nch):
            g1, g2 = gq[c]
            g1.wait()
            g2.wait()
            b1, b2 = bufs[c % nb]

            def row_comb(i, carry):
                wb1 = w1_v[c * _CCH + i, :]
                wb2 = w2_v[c * _CCH + i, :]
                for cc in range(DIM // 16):
                    sl = pl.ds(cc * 16, 16)
                    b1[i, sl] = b1[i, sl] * wb1 + b2[i, sl] * wb2
                return carry

            lax.fori_loop(0, _CCH, row_comb, 0)
            wq[c] = pltpu.async_copy(
                b1, out_hbm.at[pl.ds(base + c * _CCH, _CCH)], semw)
            if c + nb < nch:
                wq[c].wait()
                gq[c + nb] = gather(c + nb)
        for c in range(max(0, nch - nb), nch):
            wq[c].wait()

    return _combine


def kernel(inputs, gate_w, expert_w, drafting=False):
    x = inputs
    dflag = jnp.asarray(drafting, jnp.int32).reshape((1,))
    e2, w1x, w2x = _gate(x, gate_w, dflag)

    e_flat = e2.reshape(S)
    dest2d, te = _route(e_flat.reshape(32, 128))
    dest = dest2d.reshape(S)
    tile_eid = te[0, :NT]
    tile_valid = te[1, :NT]

    tok_flat = (jnp.arange(S, dtype=jnp.int32) // TOPK)
    xs = _build_dispatch()(
        x,
        tok_flat.reshape(NW, -1, _DCH),
        dest.reshape(NW, -1, _DCH),
    )

    y_s = _ffn(tile_eid, tile_valid, xs, expert_w)

    d_pair = dest.reshape(N, TOPK)
    return _build_combine()(
        y_s,
        d_pair[:, 0].reshape(NW, -1, _CCH),
        d_pair[:, 1].reshape(NW, -1, _CCH),
        w1x, w2x)
